# baseline (speedup 1.0000x reference)
import functools

import jax
import jax.numpy as jnp
from jax import lax
from jax.experimental import pallas as pl
from jax.experimental.pallas import tpu as pltpu
from jax.experimental.pallas import tpu_sc as plsc

SRC_DIM = 1024
K = 512
IN_DIM = 256
COMMIT = 0.25

N_TOTAL = 16384
BN = 1024
NW = 32
CHUNK = 32
NBUF = 3


def _tc_body(zf_ref, w_ref, b_ref, ct_ref, idx_ref, loss_ref, acc_ref, cn_ref):
    i = pl.program_id(0)

    @pl.when(i == 0)
    def _init():
        cn_ref[...] = jnp.sum(ct_ref[...] * ct_ref[...], axis=0, keepdims=True)
        acc_ref[0] = 0.0

    zp = jnp.dot(zf_ref[...], w_ref[...], preferred_element_type=jnp.float32)
    zp = zp + b_ref[...]
    g = jnp.dot(zp, ct_ref[...], preferred_element_type=jnp.float32)
    rn = jnp.sum(zp * zp, axis=1, keepdims=True)
    d = rn + cn_ref[...] - 2.0 * g
    minval = jnp.min(d, axis=1, keepdims=True)
    colid = lax.broadcasted_iota(jnp.int32, d.shape, 1).astype(jnp.float32)
    cand = jnp.where(d == minval, colid, float(K))
    idx_ref[0, 0, :] = jnp.min(cand, axis=1).astype(jnp.int32)

    acc_ref[0] += jnp.sum(minval)

    @pl.when(i == pl.num_programs(0) - 1)
    def _fini():
        scale = (1.0 + COMMIT) / (N_TOTAL * SRC_DIM)
        loss_ref[...] = jnp.full((1, 1), acc_ref[0] * scale, jnp.float32)


def _tc_distances(zf, W, b2, ct):
    n = zf.shape[0]
    nb = n // BN
    idx3, loss = pl.pallas_call(
        _tc_body,
        grid=(nb,),
        in_specs=[
            pl.BlockSpec((BN, IN_DIM), lambda i: (i, 0)),
            pl.BlockSpec((IN_DIM, SRC_DIM), lambda i: (0, 0)),
            pl.BlockSpec((1, SRC_DIM), lambda i: (0, 0)),
            pl.BlockSpec((SRC_DIM, K), lambda i: (0, 0)),
        ],
        out_specs=[
            pl.BlockSpec((1, 1, BN), lambda i: (i, 0, 0)),
            pl.BlockSpec((1, 1), lambda i: (0, 0)),
        ],
        out_shape=[
            jax.ShapeDtypeStruct((nb, 1, BN), jnp.int32),
            jax.ShapeDtypeStruct((1, 1), jnp.float32),
        ],
        scratch_shapes=[
            pltpu.SMEM((1,), jnp.float32),
            pltpu.VMEM((1, K), jnp.float32),
        ],
    )(zf, W, b2, ct)
    return idx3, loss


def _make_sc_gather(n):
    rows_per_w = n // NW
    n_iters = rows_per_w // CHUNK
    mesh = plsc.VectorSubcoreMesh(core_axis_name="c", subcore_axis_name="s")

    @functools.partial(
        pl.kernel,
        mesh=mesh,
        out_type=jax.ShapeDtypeStruct((n, SRC_DIM), jnp.float32),
        scratch_types=[
            pltpu.VMEM((rows_per_w,), jnp.int32),
            pltpu.VMEM((CHUNK, SRC_DIM), jnp.float32),
            pltpu.VMEM((CHUNK, SRC_DIM), jnp.float32),
            pltpu.VMEM((CHUNK, SRC_DIM), jnp.float32),
            pltpu.SemaphoreType.DMA,
            pltpu.SemaphoreType.DMA,
        ],
    )
    def _sc_gather(codebook_hbm, idx_hbm, out_hbm,
                   idx_v, rows0, rows1, rows2, gsem, ssem):
        wid = lax.axis_index("s") * 2 + lax.axis_index("c")
        base = wid * rows_per_w
        bufs = (rows0, rows1, rows2)
        gh = [None] * NBUF
        sh = [None] * NBUF
        pltpu.sync_copy(idx_hbm.at[pl.ds(base, CHUNK)],
                        idx_v.at[pl.ds(0, CHUNK)])
        gh[0] = pltpu.async_copy(
            codebook_hbm.at[idx_v.at[pl.ds(0, CHUNK)]], bufs[0], gsem)
        pltpu.sync_copy(idx_hbm.at[pl.ds(base + CHUNK, rows_per_w - CHUNK)],
                        idx_v.at[pl.ds(CHUNK, rows_per_w - CHUNK)])
        gh[1] = pltpu.async_copy(
            codebook_hbm.at[idx_v.at[pl.ds(CHUNK, CHUNK)]], bufs[1], gsem)
        for c in range(n_iters):
            s = c % NBUF
            gh[s].wait()
            gh[s] = None
            t = (c + 2) % NBUF
            if c + 2 < n_iters:
                if sh[t] is not None:
                    sh[t].wait()
                    sh[t] = None
                gh[t] = pltpu.async_copy(
                    codebook_hbm.at[idx_v.at[pl.ds((c + 2) * CHUNK, CHUNK)]],
                    bufs[t], gsem)
            sh[s] = pltpu.async_copy(
                bufs[s], out_hbm.at[pl.ds(base + c * CHUNK, CHUNK)], ssem)
        for s in sh:
            if s is not None:
                s.wait()

    return _sc_gather


def kernel(z, W, b, codebook):
    lead_shape = z.shape[:-1]
    zf = z.reshape(-1, IN_DIM)
    n = zf.shape[0]
    idx3, loss = _tc_distances(zf, W, b.reshape(1, SRC_DIM), codebook.T)
    idx = idx3.reshape(n)
    q = _make_sc_gather(n)(codebook, idx)
    return (
        q.reshape(lead_shape + (SRC_DIM,)),
        loss[0, 0],
        idx[:, None],
    )


# speedup vs baseline: 1.0141x; 1.0141x over previous
import functools

import jax
import jax.numpy as jnp
from jax import lax
from jax.experimental import pallas as pl
from jax.experimental.pallas import tpu as pltpu
from jax.experimental.pallas import tpu_sc as plsc

SRC_DIM = 1024
K = 512
IN_DIM = 256
COMMIT = 0.25

N_TOTAL = 16384
BN = 512
NW = 32
CHUNK = 32
NBUF = 3


def _tc_body(zf_ref, w_ref, b_ref, ct_ref, idx_ref, loss_ref, acc_ref, cn_ref):
    i = pl.program_id(0)

    @pl.when(i == 0)
    def _init():
        cn_ref[...] = jnp.sum(ct_ref[...] * ct_ref[...], axis=0, keepdims=True)
        acc_ref[0] = 0.0

    zp = jnp.dot(zf_ref[...], w_ref[...], preferred_element_type=jnp.float32)
    zp = zp + b_ref[...]
    g = jnp.dot(zp, ct_ref[...], preferred_element_type=jnp.float32)
    rn = jnp.sum(zp * zp, axis=1, keepdims=True)
    d = rn + cn_ref[...] - 2.0 * g
    minval = jnp.min(d, axis=1, keepdims=True)
    colid = lax.broadcasted_iota(jnp.int32, d.shape, 1).astype(jnp.float32)
    cand = jnp.where(d == minval, colid, float(K))
    idx_ref[0, 0, :] = jnp.min(cand, axis=1).astype(jnp.int32)

    acc_ref[0] += jnp.sum(minval)

    @pl.when(i == pl.num_programs(0) - 1)
    def _fini():
        scale = (1.0 + COMMIT) / (N_TOTAL * SRC_DIM)
        loss_ref[...] = jnp.full((1, 1), acc_ref[0] * scale, jnp.float32)


def _tc_distances(zf, W, b2, ct):
    n = zf.shape[0]
    nb = n // BN
    idx3, loss = pl.pallas_call(
        _tc_body,
        grid=(nb,),
        in_specs=[
            pl.BlockSpec((BN, IN_DIM), lambda i: (i, 0)),
            pl.BlockSpec((IN_DIM, SRC_DIM), lambda i: (0, 0)),
            pl.BlockSpec((1, SRC_DIM), lambda i: (0, 0)),
            pl.BlockSpec((SRC_DIM, K), lambda i: (0, 0)),
        ],
        out_specs=[
            pl.BlockSpec((1, 1, BN), lambda i: (i, 0, 0)),
            pl.BlockSpec((1, 1), lambda i: (0, 0)),
        ],
        out_shape=[
            jax.ShapeDtypeStruct((nb, 1, BN), jnp.int32),
            jax.ShapeDtypeStruct((1, 1), jnp.float32),
        ],
        scratch_shapes=[
            pltpu.SMEM((1,), jnp.float32),
            pltpu.VMEM((1, K), jnp.float32),
        ],
    )(zf, W, b2, ct)
    return idx3, loss


def _make_sc_gather(n):
    rows_per_w = n // NW
    n_iters = rows_per_w // CHUNK
    mesh = plsc.VectorSubcoreMesh(core_axis_name="c", subcore_axis_name="s")

    @functools.partial(
        pl.kernel,
        mesh=mesh,
        out_type=jax.ShapeDtypeStruct((n, SRC_DIM), jnp.float32),
        compiler_params=pltpu.CompilerParams(
            disable_bounds_checks=True,
            disable_semaphore_checks=True,
        ),
        scratch_types=[
            pltpu.VMEM((rows_per_w,), jnp.int32),
            pltpu.VMEM((CHUNK, SRC_DIM), jnp.float32),
            pltpu.VMEM((CHUNK, SRC_DIM), jnp.float32),
            pltpu.VMEM((CHUNK, SRC_DIM), jnp.float32),
            pltpu.SemaphoreType.DMA,
            pltpu.SemaphoreType.DMA,
        ],
    )
    def _sc_gather(codebook_hbm, idx_hbm, out_hbm,
                   idx_v, rows0, rows1, rows2, gsem, ssem):
        wid = lax.axis_index("s") * 2 + lax.axis_index("c")
        base = wid * rows_per_w
        bufs = (rows0, rows1, rows2)
        gh = [None] * NBUF
        sh = [None] * NBUF
        pltpu.sync_copy(idx_hbm.at[pl.ds(base, CHUNK)],
                        idx_v.at[pl.ds(0, CHUNK)])
        gh[0] = pltpu.async_copy(
            codebook_hbm.at[idx_v.at[pl.ds(0, CHUNK)]], bufs[0], gsem)
        pltpu.sync_copy(idx_hbm.at[pl.ds(base + CHUNK, rows_per_w - CHUNK)],
                        idx_v.at[pl.ds(CHUNK, rows_per_w - CHUNK)])
        gh[1] = pltpu.async_copy(
            codebook_hbm.at[idx_v.at[pl.ds(CHUNK, CHUNK)]], bufs[1], gsem)
        for c in range(n_iters):
            s = c % NBUF
            gh[s].wait()
            gh[s] = None
            t = (c + 2) % NBUF
            if c + 2 < n_iters:
                if sh[t] is not None:
                    sh[t].wait()
                    sh[t] = None
                gh[t] = pltpu.async_copy(
                    codebook_hbm.at[idx_v.at[pl.ds((c + 2) * CHUNK, CHUNK)]],
                    bufs[t], gsem)
            sh[s] = pltpu.async_copy(
                bufs[s], out_hbm.at[pl.ds(base + c * CHUNK, CHUNK)], ssem)
        for s in sh:
            if s is not None:
                s.wait()

    return _sc_gather


def kernel(z, W, b, codebook):
    lead_shape = z.shape[:-1]
    zf = z.reshape(-1, IN_DIM)
    n = zf.shape[0]
    idx3, loss = _tc_distances(zf, W, b.reshape(1, SRC_DIM), codebook.T)
    idx = idx3.reshape(n)
    q = _make_sc_gather(n)(codebook, idx)
    return (
        q.reshape(lead_shape + (SRC_DIM,)),
        loss[0, 0],
        idx[:, None],
    )


# speedup vs baseline: 1.0160x; 1.0019x over previous
import functools

import jax
import jax.numpy as jnp
from jax import lax
from jax.experimental import pallas as pl
from jax.experimental.pallas import tpu as pltpu
from jax.experimental.pallas import tpu_sc as plsc

SRC_DIM = 1024
K = 512
IN_DIM = 256
COMMIT = 0.25

N_TOTAL = 16384
BN = 512
NW = 32
CHUNK = 32
NBUF = 3


def _tc_body(zf_ref, w_ref, b_ref, ct_ref, idx_ref, loss_ref, acc_ref, cn_ref):
    i = pl.program_id(0)

    @pl.when(i == 0)
    def _init():
        cn_ref[...] = jnp.sum(ct_ref[...] * ct_ref[...], axis=0, keepdims=True)
        acc_ref[0] = 0.0

    zp = jnp.dot(zf_ref[...], w_ref[...], preferred_element_type=jnp.float32)
    zp = zp + b_ref[...]
    g = jnp.dot(zp, ct_ref[...], preferred_element_type=jnp.float32)
    rn = jnp.sum(zp * zp, axis=1, keepdims=True)
    d = rn + cn_ref[...] - 2.0 * g
    minval = jnp.min(d, axis=1, keepdims=True)
    colid = lax.broadcasted_iota(jnp.int32, d.shape, 1).astype(jnp.float32)
    cand = jnp.where(d == minval, colid, float(K))
    idx_ref[0, 0, :] = jnp.min(cand, axis=1).astype(jnp.int32)

    acc_ref[0] += jnp.sum(minval)

    @pl.when(i == pl.num_programs(0) - 1)
    def _fini():
        scale = (1.0 + COMMIT) / (N_TOTAL * SRC_DIM)
        loss_ref[...] = jnp.full((1, 1), acc_ref[0] * scale, jnp.float32)


def _tc_distances(zf, W, b2, ct):
    n = zf.shape[0]
    nb = n // BN
    idx3, loss = pl.pallas_call(
        _tc_body,
        grid=(nb,),
        in_specs=[
            pl.BlockSpec((BN, IN_DIM), lambda i: (i, 0)),
            pl.BlockSpec((IN_DIM, SRC_DIM), lambda i: (0, 0)),
            pl.BlockSpec((1, SRC_DIM), lambda i: (0, 0)),
            pl.BlockSpec((SRC_DIM, K), lambda i: (0, 0)),
        ],
        out_specs=[
            pl.BlockSpec((1, 1, BN), lambda i: (i, 0, 0)),
            pl.BlockSpec((1, 1), lambda i: (0, 0)),
        ],
        out_shape=[
            jax.ShapeDtypeStruct((nb, 1, BN), jnp.int32),
            jax.ShapeDtypeStruct((1, 1), jnp.float32),
        ],
        scratch_shapes=[
            pltpu.SMEM((1,), jnp.float32),
            pltpu.VMEM((1, K), jnp.float32),
        ],
    )(zf, W, b2, ct)
    return idx3, loss


def _make_sc_gather(n):
    rows_per_w = n // NW
    n_iters = rows_per_w // CHUNK
    mesh = plsc.VectorSubcoreMesh(core_axis_name="c", subcore_axis_name="s")

    @functools.partial(
        pl.kernel,
        mesh=mesh,
        out_type=jax.ShapeDtypeStruct((n, SRC_DIM), jnp.float32),
        scratch_types=[
            pltpu.VMEM((rows_per_w,), jnp.int32),
            pltpu.VMEM((CHUNK, SRC_DIM), jnp.float32),
            pltpu.VMEM((CHUNK, SRC_DIM), jnp.float32),
            pltpu.VMEM((CHUNK, SRC_DIM), jnp.float32),
            pltpu.SemaphoreType.DMA,
            pltpu.SemaphoreType.DMA,
        ],
    )
    def _sc_gather(codebook_hbm, idx_hbm, out_hbm,
                   idx_v, rows0, rows1, rows2, gsem, ssem):
        wid = lax.axis_index("s") * 2 + lax.axis_index("c")
        base = wid * rows_per_w
        bufs = (rows0, rows1, rows2)
        gh = [None] * NBUF
        sh = [None] * NBUF
        pltpu.sync_copy(idx_hbm.at[pl.ds(base, CHUNK)],
                        idx_v.at[pl.ds(0, CHUNK)])
        gh[0] = pltpu.async_copy(
            codebook_hbm.at[idx_v.at[pl.ds(0, CHUNK)]], bufs[0], gsem)
        pltpu.sync_copy(idx_hbm.at[pl.ds(base + CHUNK, rows_per_w - CHUNK)],
                        idx_v.at[pl.ds(CHUNK, rows_per_w - CHUNK)])
        gh[1] = pltpu.async_copy(
            codebook_hbm.at[idx_v.at[pl.ds(CHUNK, CHUNK)]], bufs[1], gsem)
        for c in range(n_iters):
            s = c % NBUF
            gh[s].wait()
            gh[s] = None
            t = (c + 2) % NBUF
            if c + 2 < n_iters:
                if sh[t] is not None:
                    sh[t].wait()
                    sh[t] = None
                gh[t] = pltpu.async_copy(
                    codebook_hbm.at[idx_v.at[pl.ds((c + 2) * CHUNK, CHUNK)]],
                    bufs[t], gsem)
            sh[s] = pltpu.async_copy(
                bufs[s], out_hbm.at[pl.ds(base + c * CHUNK, CHUNK)], ssem)
        for s in sh:
            if s is not None:
                s.wait()

    return _sc_gather


def kernel(z, W, b, codebook):
    lead_shape = z.shape[:-1]
    zf = z.reshape(-1, IN_DIM)
    n = zf.shape[0]
    idx3, loss = _tc_distances(zf, W, b.reshape(1, SRC_DIM), codebook.T)
    idx = idx3.reshape(n)
    q = _make_sc_gather(n)(codebook, idx)
    return (
        q.reshape(lead_shape + (SRC_DIM,)),
        loss[0, 0],
        idx[:, None],
    )
